# baseline (speedup 1.0000x reference)
import functools

import jax
import jax.numpy as jnp
from jax import lax
from jax.experimental import pallas as pl
from jax.experimental.pallas import tpu as pltpu
from jax.experimental.pallas import tpu_sc as plsc

_VOCAB = 1000000
_EMB_DIM = 64
_KNOW_DIM = 32
_OUT_DIM = _EMB_DIM + _KNOW_DIM
_BATCH = 4096
_HIST = 50

_N = _BATCH * _HIST
_NW = 32
_BPW = _BATCH // _NW
_BPC = 2
_CHUNK = _BPC * _HIST
_NCHUNK = _BPW // _BPC
_GCHUNK = _N // _CHUNK
_LANES = 128

_VMAIN = 999936
_WBLK = 128
_NWB = _VMAIN // _WBLK
_KBLK = 128
_NKB = _VMAIN // _KBLK
_TAIL = 128
_TSTART = _VOCAB - _TAIL


def _iota16():
    return jnp.arange(16, dtype=jnp.int32)


def _tpose_body(wt_hbm, kt_hbm, wtl_hbm, ktl_hbm, wv_hbm, kv_hbm,
                slab_w, buf_w, slab_k, buf_k):
    nc = 2
    wid = lax.axis_index("s") * nc + lax.axis_index("c")

    def word_block(wb, carry):
        bid = wid + _NW * wb

        @pl.when(bid < _NWB)
        def _():
            c0 = bid * _WBLK
            pltpu.sync_copy(wt_hbm.at[:, pl.ds(c0, _WBLK)], slab_w)

            def row(r, c2):
                for half in range(2):
                    cidx = jnp.broadcast_to(2 * r + half, (16,)).astype(
                        jnp.int32)
                    for g in range(4):
                        vals = plsc.load_gather(
                            slab_w, [_iota16() + 16 * g, cidx])
                        buf_w[r, pl.ds(64 * half + 16 * g, 16)] = vals
                return c2

            lax.fori_loop(0, _WBLK // 2, row, 0)
            pltpu.sync_copy(buf_w, wv_hbm.at[pl.ds(bid * (_WBLK // 2),
                                                   _WBLK // 2)])
        return carry

    lax.fori_loop(0, (_NWB + _NW - 1) // _NW, word_block, 0)

    def know_block(kb, carry):
        bid = wid + _NW * kb

        @pl.when(bid < _NKB)
        def _():
            c0 = bid * _KBLK
            pltpu.sync_copy(kt_hbm.at[:, pl.ds(c0, _KBLK)], slab_k)

            def row(r, c2):
                for g in range(8):
                    q, fh = g // 2, g % 2
                    cidx = jnp.broadcast_to(4 * r + q, (16,)).astype(
                        jnp.int32)
                    vals = plsc.load_gather(
                        slab_k, [_iota16() + 16 * fh, cidx])
                    buf_k[r, pl.ds(16 * g, 16)] = vals
                return c2

            lax.fori_loop(0, _KBLK // 4, row, 0)
            pltpu.sync_copy(buf_k, kv_hbm.at[pl.ds(bid * (_KBLK // 4),
                                                   _KBLK // 4)])
        return carry

    lax.fori_loop(0, (_NKB + _NW - 1) // _NW, know_block, 0)

    @pl.when(wid == 0)
    def _():
        pltpu.sync_copy(wtl_hbm, slab_w.at[:, pl.ds(0, _TAIL)])

        def row(r, c2):
            for half in range(2):
                cidx = jnp.broadcast_to(2 * r + half, (16,)).astype(jnp.int32)
                for g in range(4):
                    vals = plsc.load_gather(
                        slab_w, [_iota16() + 16 * g, cidx])
                    buf_w[r, pl.ds(64 * half + 16 * g, 16)] = vals
            return c2

        lax.fori_loop(0, _TAIL // 2, row, 0)
        pltpu.sync_copy(buf_w.at[pl.ds(0, _TAIL // 2)],
                        wv_hbm.at[pl.ds(_TSTART // 2, _TAIL // 2)])

    @pl.when(wid == 1)
    def _():
        pltpu.sync_copy(ktl_hbm, slab_k.at[:, pl.ds(0, _TAIL)])

        def row(r, c2):
            for g in range(8):
                q, fh = g // 2, g % 2
                cidx = jnp.broadcast_to(4 * r + q, (16,)).astype(jnp.int32)
                vals = plsc.load_gather(slab_k, [_iota16() + 16 * fh, cidx])
                buf_k[r, pl.ds(16 * g, 16)] = vals
            return c2

        lax.fori_loop(0, _TAIL // 4, row, 0)
        pltpu.sync_copy(buf_k.at[pl.ds(0, _TAIL // 4)],
                        kv_hbm.at[pl.ds(_TSTART // 4, _TAIL // 4)])


def _sc_body(xp_hbm, xw_hbm, xk_hbm, word_hbm, know_hbm, out_hbm,
             xp_v, xw_v, xk_v, wp_a, wp_b, kq_a, kq_b, st_a, st_b,
             sw_a, sw_b, sk_a, sk_b):
    nc = 2
    wid = lax.axis_index("s") * nc + lax.axis_index("c")
    crow0 = wid * _NCHUNK
    pltpu.sync_copy(xp_hbm.at[pl.ds(crow0, _NCHUNK)], xp_v)
    pltpu.sync_copy(xw_hbm.at[pl.ds(crow0, _NCHUNK)], xw_v)
    pltpu.sync_copy(xk_hbm.at[pl.ds(crow0, _NCHUNK)], xk_v)

    bufs = ((wp_a, kq_a, st_a, sw_a, sk_a),
            (wp_b, kq_b, st_b, sw_b, sk_b))

    def fire(c, wp, kq, sw, sk):
        pltpu.async_copy(word_hbm.at[xw_v.at[c]], wp, sw)
        pltpu.async_copy(know_hbm.at[xk_v.at[c]], kq, sk)

    fire(0, wp_a, kq_a, sw_a, sk_a)
    fire(1, wp_b, kq_b, sw_b, sk_b)

    def gloop(g, carry):
        for b in (0, 1):
            wp, kq, st, sw, sk = bufs[b]
            c = 2 * g + b
            pltpu.make_async_copy(word_hbm.at[xw_v.at[c]], wp, sw).wait()
            pltpu.make_async_copy(know_hbm.at[xk_v.at[c]], kq, sk).wait()

            def movegrp(bb, hbase, nrows, xvec):
                for t in range(nrows):
                    row = bb * _HIST + hbase + t
                    idx = xvec[t]
                    h = (idx & 1) * _EMB_DIM
                    q = (idx & 3) * _KNOW_DIM
                    for k in range(4):
                        st[bb, hbase + t, pl.ds(16 * k, 16)] = (
                            wp[row, pl.ds(h + 16 * k, 16)])
                    for k in range(2):
                        st[bb, hbase + t, pl.ds(_EMB_DIM + 16 * k, 16)] = (
                            kq[row, pl.ds(q + 16 * k, 16)])

            for bb in (0, 1):
                def mloop(gg, c2, bb=bb):
                    movegrp(bb, gg * 16, 16,
                            xp_v[c, pl.ds(bb * _HIST + gg * 16, 16)])
                    return c2

                lax.fori_loop(0, _HIST // 16, mloop, 0)
                movegrp(bb, 48, 2, xp_v[c, pl.ds(bb * _HIST + 48, 16)])

            pltpu.sync_copy(
                st, out_hbm.at[pl.ds(wid * _BPW + c * _BPC, _BPC)])

            @pl.when(c + 2 < _NCHUNK)
            def _():
                fire(c + 2, wp, kq, sw, sk)
        return carry

    lax.fori_loop(0, _NCHUNK // 2, gloop, 0)


@jax.jit
def _lookup(xp2, xw2, xk2, word_t, know_t, word_tail, know_tail):
    mesh = plsc.VectorSubcoreMesh(core_axis_name="c", subcore_axis_name="s")
    word_view, know_view = pl.kernel(
        _tpose_body,
        out_type=(jax.ShapeDtypeStruct((_VOCAB // 2, 2 * _EMB_DIM),
                                       jnp.float32),
                  jax.ShapeDtypeStruct((_VOCAB // 4, 4 * _KNOW_DIM),
                                       jnp.float32)),
        mesh=mesh,
        scratch_types=[
            pltpu.VMEM((_EMB_DIM, _WBLK), jnp.float32),
            pltpu.VMEM((_WBLK // 2, _LANES), jnp.float32),
            pltpu.VMEM((_KNOW_DIM, _KBLK), jnp.float32),
            pltpu.VMEM((_KBLK // 4, _LANES), jnp.float32),
        ],
        compiler_params=pltpu.CompilerParams(needs_layout_passes=False),
    )(word_t, know_t, word_tail, know_tail)

    return pl.kernel(
        _sc_body,
        out_type=jax.ShapeDtypeStruct((_BATCH, _HIST, _OUT_DIM),
                                      jnp.float32),
        mesh=mesh,
        scratch_types=[
            pltpu.VMEM((_NCHUNK, _LANES), jnp.int32),
            pltpu.VMEM((_NCHUNK, _LANES), jnp.int32),
            pltpu.VMEM((_NCHUNK, _LANES), jnp.int32),
            pltpu.VMEM((_LANES, _LANES), jnp.float32),
            pltpu.VMEM((_LANES, _LANES), jnp.float32),
            pltpu.VMEM((_LANES, _LANES), jnp.float32),
            pltpu.VMEM((_LANES, _LANES), jnp.float32),
            pltpu.VMEM((_BPC, _HIST, _OUT_DIM), jnp.float32),
            pltpu.VMEM((_BPC, _HIST, _OUT_DIM), jnp.float32),
            pltpu.SemaphoreType.DMA,
            pltpu.SemaphoreType.DMA,
            pltpu.SemaphoreType.DMA,
            pltpu.SemaphoreType.DMA,
        ],
    )(xp2, xw2, xk2, word_view, know_view)


def kernel(x, word_table, knowledge_table):
    x1d = x.astype(jnp.int32).reshape(_N)
    xpad = jnp.pad(x1d.reshape(_GCHUNK, _CHUNK),
                   ((0, 0), (0, _LANES - _CHUNK)), mode="edge")
    word_t = word_table.T
    know_t = knowledge_table.T
    return _lookup(xpad, xpad >> 1, xpad >> 2, word_t, know_t,
                   word_t[:, _TSTART:], know_t[:, _TSTART:])


# speedup vs baseline: 1.2290x; 1.2290x over previous
import functools

import jax
import jax.numpy as jnp
from jax import lax
from jax.experimental import pallas as pl
from jax.experimental.pallas import tpu as pltpu
from jax.experimental.pallas import tpu_sc as plsc

_VOCAB = 1000000
_EMB_DIM = 64
_KNOW_DIM = 32
_OUT_DIM = _EMB_DIM + _KNOW_DIM
_BATCH = 4096
_HIST = 50

_N = _BATCH * _HIST
_NW = 32
_BPW = _BATCH // _NW
_BPC = 2
_CHUNK = _BPC * _HIST
_NCHUNK = _BPW // _BPC
_GCHUNK = _N // _CHUNK
_LANES = 128

_VMAIN = 999936
_WBLK = 128
_NWB = _VMAIN // _WBLK
_KBLK = 128
_NKB = _VMAIN // _KBLK
_TAIL = 128
_TSTART = _VOCAB - _TAIL


def _iota16():
    return jnp.arange(16, dtype=jnp.int32)


def _tpose_body(wt_hbm, kt_hbm, wtl_hbm, ktl_hbm, wv_hbm, kv_hbm,
                slab_wa, slab_wb, buf_wa, buf_wb,
                slab_ka, slab_kb, buf_ka, buf_kb,
                siw_a, siw_b, sow_a, sow_b,
                sik_a, sik_b, sok_a, sok_b):
    nc = 2
    wid = lax.axis_index("s") * nc + lax.axis_index("c")

    def _table_loop(src_hbm, dst_hbm, nblk, rows_fn, nrows,
                    slabs, bufs, sis, sos, blkw):
        def fire_in(wb, slab, si):
            bid = wid + _NW * wb

            @pl.when(bid < nblk)
            def _():
                pltpu.async_copy(src_hbm.at[:, pl.ds(bid * blkw, blkw)],
                                 slab, si)

        fire_in(0, slabs[0], sis[0])
        fire_in(1, slabs[1], sis[1])

        def pair(gg, carry):
            for b in (0, 1):
                wb = 2 * gg + b
                bid = wid + _NW * wb

                @pl.when(bid < nblk)
                def _(b=b, wb=wb, bid=bid):
                    slab, buf, si, so = slabs[b], bufs[b], sis[b], sos[b]
                    pltpu.make_async_copy(
                        src_hbm.at[:, pl.ds(bid * blkw, blkw)], slab,
                        si).wait()

                    @pl.when(wb >= 2)
                    def _():
                        pltpu.make_async_copy(
                            buf, dst_hbm.at[pl.ds(0, nrows)], so).wait()

                    lax.fori_loop(0, nrows,
                                  lambda r, c2: rows_fn(r, slab, buf, c2),
                                  0)
                    pltpu.async_copy(buf,
                                     dst_hbm.at[pl.ds(bid * nrows, nrows)],
                                     so)
                    fire_in(wb + 2, slab, si)
            return carry

        npair = ((nblk + _NW - 1) // _NW + 3) // 2
        lax.fori_loop(0, npair, pair, 0)
        for b in (0, 1):
            pltpu.make_async_copy(bufs[b], dst_hbm.at[pl.ds(0, nrows)],
                                  sos[b]).wait()

    def word_rows(r, slab, buf, c2):
        for half in range(2):
            cidx = jnp.broadcast_to(2 * r + half, (16,)).astype(jnp.int32)
            for g in range(4):
                vals = plsc.load_gather(slab, [_iota16() + 16 * g, cidx])
                buf[r, pl.ds(64 * half + 16 * g, 16)] = vals
        return c2

    def know_rows(r, slab, buf, c2):
        for g in range(8):
            q, fh = g // 2, g % 2
            cidx = jnp.broadcast_to(4 * r + q, (16,)).astype(jnp.int32)
            vals = plsc.load_gather(slab, [_iota16() + 16 * fh, cidx])
            buf[r, pl.ds(16 * g, 16)] = vals
        return c2

    _table_loop(wt_hbm, wv_hbm, _NWB, word_rows, _WBLK // 2,
                (slab_wa, slab_wb), (buf_wa, buf_wb),
                (siw_a, siw_b), (sow_a, sow_b), _WBLK)
    _table_loop(kt_hbm, kv_hbm, _NKB, know_rows, _KBLK // 4,
                (slab_ka, slab_kb), (buf_ka, buf_kb),
                (sik_a, sik_b), (sok_a, sok_b), _KBLK)

    @pl.when(wid == 0)
    def _():
        pltpu.sync_copy(wtl_hbm, slab_wa.at[:, pl.ds(0, _TAIL)])

        def row(r, c2):
            for half in range(2):
                cidx = jnp.broadcast_to(2 * r + half, (16,)).astype(jnp.int32)
                for g in range(4):
                    vals = plsc.load_gather(
                        slab_wa, [_iota16() + 16 * g, cidx])
                    buf_wa[r, pl.ds(64 * half + 16 * g, 16)] = vals
            return c2

        lax.fori_loop(0, _TAIL // 2, row, 0)
        pltpu.sync_copy(buf_wa.at[pl.ds(0, _TAIL // 2)],
                        wv_hbm.at[pl.ds(_TSTART // 2, _TAIL // 2)])

    @pl.when(wid == 1)
    def _():
        pltpu.sync_copy(ktl_hbm, slab_ka.at[:, pl.ds(0, _TAIL)])

        def row(r, c2):
            for g in range(8):
                q, fh = g // 2, g % 2
                cidx = jnp.broadcast_to(4 * r + q, (16,)).astype(jnp.int32)
                vals = plsc.load_gather(slab_ka, [_iota16() + 16 * fh, cidx])
                buf_ka[r, pl.ds(16 * g, 16)] = vals
            return c2

        lax.fori_loop(0, _TAIL // 4, row, 0)
        pltpu.sync_copy(buf_ka.at[pl.ds(0, _TAIL // 4)],
                        kv_hbm.at[pl.ds(_TSTART // 4, _TAIL // 4)])


def _sc_body(xp_hbm, xw_hbm, xk_hbm, word_hbm, know_hbm, out_hbm,
             xp_v, xw_v, xk_v, wp_a, wp_b, kq_a, kq_b, st_a, st_b,
             sw_a, sw_b, sk_a, sk_b):
    nc = 2
    wid = lax.axis_index("s") * nc + lax.axis_index("c")
    crow0 = wid * _NCHUNK
    pltpu.sync_copy(xp_hbm.at[pl.ds(crow0, _NCHUNK)], xp_v)
    pltpu.sync_copy(xw_hbm.at[pl.ds(crow0, _NCHUNK)], xw_v)
    pltpu.sync_copy(xk_hbm.at[pl.ds(crow0, _NCHUNK)], xk_v)

    bufs = ((wp_a, kq_a, st_a, sw_a, sk_a),
            (wp_b, kq_b, st_b, sw_b, sk_b))

    def fire(c, wp, kq, sw, sk):
        pltpu.async_copy(word_hbm.at[xw_v.at[c]], wp, sw)
        pltpu.async_copy(know_hbm.at[xk_v.at[c]], kq, sk)

    fire(0, wp_a, kq_a, sw_a, sk_a)
    fire(1, wp_b, kq_b, sw_b, sk_b)

    def gloop(g, carry):
        for b in (0, 1):
            wp, kq, st, sw, sk = bufs[b]
            c = 2 * g + b
            pltpu.make_async_copy(word_hbm.at[xw_v.at[c]], wp, sw).wait()
            pltpu.make_async_copy(know_hbm.at[xk_v.at[c]], kq, sk).wait()

            def movegrp(bb, hbase, nrows, xvec):
                for t in range(nrows):
                    row = bb * _HIST + hbase + t
                    idx = xvec[t]
                    h = (idx & 1) * _EMB_DIM
                    q = (idx & 3) * _KNOW_DIM
                    for k in range(4):
                        st[bb, hbase + t, pl.ds(16 * k, 16)] = (
                            wp[row, pl.ds(h + 16 * k, 16)])
                    for k in range(2):
                        st[bb, hbase + t, pl.ds(_EMB_DIM + 16 * k, 16)] = (
                            kq[row, pl.ds(q + 16 * k, 16)])

            for bb in (0, 1):
                def mloop(gg, c2, bb=bb):
                    movegrp(bb, gg * 16, 16,
                            xp_v[c, pl.ds(bb * _HIST + gg * 16, 16)])
                    return c2

                lax.fori_loop(0, _HIST // 16, mloop, 0)
                movegrp(bb, 48, 2, xp_v[c, pl.ds(bb * _HIST + 48, 16)])

            pltpu.sync_copy(
                st, out_hbm.at[pl.ds(wid * _BPW + c * _BPC, _BPC)])

            @pl.when(c + 2 < _NCHUNK)
            def _():
                fire(c + 2, wp, kq, sw, sk)
        return carry

    lax.fori_loop(0, _NCHUNK // 2, gloop, 0)


@jax.jit
def _lookup(xp2, xw2, xk2, word_t, know_t, word_tail, know_tail):
    mesh = plsc.VectorSubcoreMesh(core_axis_name="c", subcore_axis_name="s")
    word_view, know_view = pl.kernel(
        _tpose_body,
        out_type=(jax.ShapeDtypeStruct((_VOCAB // 2, 2 * _EMB_DIM),
                                       jnp.float32),
                  jax.ShapeDtypeStruct((_VOCAB // 4, 4 * _KNOW_DIM),
                                       jnp.float32)),
        mesh=mesh,
        scratch_types=[
            pltpu.VMEM((_EMB_DIM, _WBLK), jnp.float32),
            pltpu.VMEM((_EMB_DIM, _WBLK), jnp.float32),
            pltpu.VMEM((_WBLK // 2, _LANES), jnp.float32),
            pltpu.VMEM((_WBLK // 2, _LANES), jnp.float32),
            pltpu.VMEM((_KNOW_DIM, _KBLK), jnp.float32),
            pltpu.VMEM((_KNOW_DIM, _KBLK), jnp.float32),
            pltpu.VMEM((_KBLK // 4, _LANES), jnp.float32),
            pltpu.VMEM((_KBLK // 4, _LANES), jnp.float32),
            pltpu.SemaphoreType.DMA,
            pltpu.SemaphoreType.DMA,
            pltpu.SemaphoreType.DMA,
            pltpu.SemaphoreType.DMA,
            pltpu.SemaphoreType.DMA,
            pltpu.SemaphoreType.DMA,
            pltpu.SemaphoreType.DMA,
            pltpu.SemaphoreType.DMA,
        ],
        compiler_params=pltpu.CompilerParams(needs_layout_passes=False),
    )(word_t, know_t, word_tail, know_tail)

    return pl.kernel(
        _sc_body,
        out_type=jax.ShapeDtypeStruct((_BATCH, _HIST, _OUT_DIM),
                                      jnp.float32),
        mesh=mesh,
        scratch_types=[
            pltpu.VMEM((_NCHUNK, _LANES), jnp.int32),
            pltpu.VMEM((_NCHUNK, _LANES), jnp.int32),
            pltpu.VMEM((_NCHUNK, _LANES), jnp.int32),
            pltpu.VMEM((_LANES, _LANES), jnp.float32),
            pltpu.VMEM((_LANES, _LANES), jnp.float32),
            pltpu.VMEM((_LANES, _LANES), jnp.float32),
            pltpu.VMEM((_LANES, _LANES), jnp.float32),
            pltpu.VMEM((_BPC, _HIST, _OUT_DIM), jnp.float32),
            pltpu.VMEM((_BPC, _HIST, _OUT_DIM), jnp.float32),
            pltpu.SemaphoreType.DMA,
            pltpu.SemaphoreType.DMA,
            pltpu.SemaphoreType.DMA,
            pltpu.SemaphoreType.DMA,
        ],
    )(xp2, xw2, xk2, word_view, know_view)


def kernel(x, word_table, knowledge_table):
    x1d = x.astype(jnp.int32).reshape(_N)
    xpad = jnp.pad(x1d.reshape(_GCHUNK, _CHUNK),
                   ((0, 0), (0, _LANES - _CHUNK)), mode="edge")
    word_t = word_table.T
    know_t = knowledge_table.T
    return _lookup(xpad, xpad >> 1, xpad >> 2, word_t, know_t,
                   word_t[:, _TSTART:], know_t[:, _TSTART:])
